# baseline (speedup 1.0000x reference)
import functools

import jax
import jax.numpy as jnp
from jax import lax
from jax.experimental import pallas as pl
from jax.experimental.pallas import tpu as pltpu
from jax.experimental.pallas import tpu_sc as plsc


def _sc_geometry():
    try:
        info = plsc.get_sparse_core_info()
        return info.num_cores, info.num_subcores
    except Exception:
        return 2, 16


@functools.lru_cache(maxsize=None)
def _make_gather(B, V, D):
    NC, NS = _sc_geometry()
    NW = NC * NS
    assert B % NW == 0
    b_per_w = B // NW
    CB = 128
    WINDOW = 64

    mesh = plsc.VectorSubcoreMesh(core_axis_name="c", subcore_axis_name="s")

    @functools.partial(
        pl.kernel,
        out_type=(
            jax.ShapeDtypeStruct((B, D), jnp.float32),
            jax.ShapeDtypeStruct((B,), jnp.float32),
        ),
        mesh=mesh,
        scratch_types=[
            pltpu.VMEM((b_per_w,), jnp.int32),
            pltpu.VMEM((b_per_w,), jnp.float32),
            pltpu.SemaphoreType.DMA,
            pltpu.SemaphoreType.DMA,
        ],
    )
    def k(ids_hbm, emb_hbm, bias_hbm, out_emb, out_bias,
          idx_v, bias_v, sem_d, sem_b):
        wid = lax.axis_index("s") * NC + lax.axis_index("c")
        base = wid * b_per_w

        pltpu.sync_copy(ids_hbm.at[pl.ds(base, b_per_w)], idx_v)

        for c in range(b_per_w // CB):
            pltpu.async_copy(bias_hbm.at[idx_v.at[pl.ds(c * CB, CB)]],
                             bias_v.at[pl.ds(c * CB, CB)], sem_b)

        def row_wait():
            pltpu.make_async_copy(emb_hbm.at[pl.ds(0, 1)],
                                  out_emb.at[pl.ds(base, 1)], sem_d).wait()

        n_grp = b_per_w // 16
        wgrp = WINDOW // 16

        def issue(j, carry):
            vec = idx_v[pl.ds(j * 16, 16)]
            for kk in range(16):
                pltpu.make_async_copy(
                    emb_hbm.at[pl.ds(vec[kk], 1)],
                    out_emb.at[pl.ds(base + j * 16 + kk, 1)], sem_d).start()

            @pl.when(j >= wgrp)
            def _():
                for kk in range(16):
                    row_wait()
            return carry

        lax.fori_loop(0, n_grp, issue, 0)
        for _ in range(min(wgrp, n_grp) * 16):
            row_wait()

        for c in range(b_per_w // CB):
            pltpu.make_async_copy(bias_hbm.at[idx_v.at[pl.ds(c * CB, CB)]],
                                  bias_v.at[pl.ds(c * CB, CB)], sem_b).wait()
        pltpu.sync_copy(bias_v, out_bias.at[pl.ds(base, b_per_w)])

    return k


def kernel(target_surface_forms, target_priors, input_embeddings, bias):
    B = target_surface_forms.shape[0]
    V, D = input_embeddings.shape
    ids = target_surface_forms[:, 0].astype(jnp.int32)
    gather = _make_gather(B, V, D)
    out_emb, out_bias = gather(ids, input_embeddings, bias.reshape(V))
    return (out_emb, out_bias)


# speedup vs baseline: 21.7717x; 21.7717x over previous
import functools

import jax
import jax.numpy as jnp
from jax import lax
from jax.experimental import pallas as pl
from jax.experimental.pallas import tpu as pltpu
from jax.experimental.pallas import tpu_sc as plsc


def _sc_geometry():
    try:
        info = plsc.get_sparse_core_info()
        return info.num_cores, info.num_subcores
    except Exception:
        return 2, 16


@functools.lru_cache(maxsize=None)
def _make_gather(B, V, D, L):
    NC, NS = _sc_geometry()
    NW = NC * NS
    assert B % NW == 0
    b_per_w = B // NW
    C = 32
    assert b_per_w % C == 0
    n_chunks = b_per_w // C
    NBUF = 4

    mesh = plsc.VectorSubcoreMesh(core_axis_name="c", subcore_axis_name="s")

    @functools.partial(
        pl.kernel,
        out_type=(
            jax.ShapeDtypeStruct((B, D), jnp.float32),
            jax.ShapeDtypeStruct((B,), jnp.float32),
        ),
        mesh=mesh,
        scratch_types=[
            pltpu.VMEM((b_per_w * L,), jnp.int32),
            pltpu.VMEM((b_per_w,), jnp.int32),
            pltpu.VMEM((NBUF, C, D), jnp.float32),
            pltpu.VMEM((b_per_w,), jnp.float32),
            pltpu.SemaphoreType.DMA,
            pltpu.SemaphoreType.DMA,
            pltpu.SemaphoreType.DMA,
        ],
    )
    def k(tsf_hbm, emb_hbm, bias_hbm, out_emb, out_bias,
          tsf_v, idx_v, rows_v, bias_v, sem_r, sem_w, sem_b):
        wid = lax.axis_index("s") * NC + lax.axis_index("c")
        base = wid * b_per_w

        pltpu.sync_copy(tsf_hbm.at[pl.ds(base * L, b_per_w * L)], tsf_v)
        lanes = lax.iota(jnp.int32, 16)
        acc = jnp.zeros((16,), jnp.int32)
        for j in range(b_per_w):
            vec = tsf_v[pl.ds(j * L, 16)]
            acc = jnp.where(lanes == (j % 16), vec[0], acc)
            if j % 16 == 15:
                idx_v[pl.ds(j - 15, 16)] = acc

        def gather(g, buf):
            return pltpu.make_async_copy(
                emb_hbm.at[idx_v.at[pl.ds(g * C, C)]], rows_v.at[buf], sem_r)

        def write(g, buf):
            return pltpu.make_async_copy(
                rows_v.at[buf], out_emb.at[pl.ds(base + g * C, C)], sem_w)

        CB = 128
        for c in range(b_per_w // CB):
            pltpu.async_copy(bias_hbm.at[idx_v.at[pl.ds(c * CB, CB)]],
                             bias_v.at[pl.ds(c * CB, CB)], sem_b)

        for g in range(min(NBUF, n_chunks)):
            gather(g, g % NBUF).start()
        for g in range(n_chunks):
            buf = g % NBUF
            gather(g, buf).wait()
            write(g, buf).start()
            if g + NBUF < n_chunks:
                write(g, buf).wait()
                gather(g + NBUF, buf).start()
        for g in range(max(0, n_chunks - NBUF), n_chunks):
            write(g, g % NBUF).wait()

        for c in range(b_per_w // CB):
            pltpu.make_async_copy(bias_hbm.at[idx_v.at[pl.ds(c * CB, CB)]],
                                  bias_v.at[pl.ds(c * CB, CB)], sem_b).wait()
        pltpu.sync_copy(bias_v, out_bias.at[pl.ds(base, b_per_w)])

    return k


def kernel(target_surface_forms, target_priors, input_embeddings, bias):
    B = target_surface_forms.shape[0]
    V, D = input_embeddings.shape
    L = target_surface_forms.shape[1]
    gather = _make_gather(B, V, D, L)
    tsf_flat = target_surface_forms.astype(jnp.int32).reshape(B * L)
    out_emb, out_bias = gather(tsf_flat, input_embeddings, bias.reshape(V))
    return (out_emb, out_bias)


# speedup vs baseline: 26.8196x; 1.2319x over previous
import functools

import jax
import jax.numpy as jnp
from jax import lax
from jax.experimental import pallas as pl
from jax.experimental.pallas import tpu as pltpu
from jax.experimental.pallas import tpu_sc as plsc


def _sc_geometry():
    try:
        info = plsc.get_sparse_core_info()
        return info.num_cores, info.num_subcores
    except Exception:
        return 2, 16


@functools.lru_cache(maxsize=None)
def _make_gather(B, V, D):
    NC, NS = _sc_geometry()
    NW = NC * NS
    assert B % NW == 0
    b_per_w = B // NW
    C = 64
    assert b_per_w % C == 0
    n_chunks = b_per_w // C
    NBUF = 2

    mesh = plsc.VectorSubcoreMesh(core_axis_name="c", subcore_axis_name="s")

    @functools.partial(
        pl.kernel,
        out_type=(
            jax.ShapeDtypeStruct((B, D), jnp.float32),
            jax.ShapeDtypeStruct((B,), jnp.float32),
        ),
        mesh=mesh,
        scratch_types=[
            pltpu.VMEM((b_per_w,), jnp.int32),
            pltpu.VMEM((NBUF, C, D), jnp.float32),
            pltpu.VMEM((b_per_w,), jnp.float32),
            pltpu.SemaphoreType.DMA,
            pltpu.SemaphoreType.DMA,
            pltpu.SemaphoreType.DMA,
        ],
    )
    def k(ids_hbm, emb_hbm, bias_hbm, out_emb, out_bias,
          idx_v, rows_v, bias_v, sem_r, sem_w, sem_b):
        wid = lax.axis_index("s") * NC + lax.axis_index("c")
        base = wid * b_per_w

        pltpu.sync_copy(ids_hbm.at[pl.ds(base, b_per_w)], idx_v)

        def gather(g, buf):
            return pltpu.make_async_copy(
                emb_hbm.at[idx_v.at[pl.ds(g * C, C)]], rows_v.at[buf], sem_r)

        def write(g, buf):
            return pltpu.make_async_copy(
                rows_v.at[buf], out_emb.at[pl.ds(base + g * C, C)], sem_w)

        CB = 128
        for c in range(b_per_w // CB):
            pltpu.async_copy(bias_hbm.at[idx_v.at[pl.ds(c * CB, CB)]],
                             bias_v.at[pl.ds(c * CB, CB)], sem_b)

        for g in range(min(NBUF, n_chunks)):
            gather(g, g % NBUF).start()
        for g in range(n_chunks):
            buf = g % NBUF
            gather(g, buf).wait()
            write(g, buf).start()
            if g + NBUF < n_chunks:
                write(g, buf).wait()
                gather(g + NBUF, buf).start()
        for c in range(b_per_w // CB):
            pltpu.make_async_copy(bias_hbm.at[idx_v.at[pl.ds(c * CB, CB)]],
                                  bias_v.at[pl.ds(c * CB, CB)], sem_b).wait()
        pltpu.async_copy(bias_v, out_bias.at[pl.ds(base, b_per_w)], sem_b)
        for g in range(max(0, n_chunks - NBUF), n_chunks):
            write(g, g % NBUF).wait()
        pltpu.make_async_copy(bias_v, out_bias.at[pl.ds(base, b_per_w)],
                              sem_b).wait()

    return k


def kernel(target_surface_forms, target_priors, input_embeddings, bias):
    B = target_surface_forms.shape[0]
    V, D = input_embeddings.shape
    ids = target_surface_forms[:, 0].astype(jnp.int32)
    gather = _make_gather(B, V, D)
    out_emb, out_bias = gather(ids, input_embeddings, bias.reshape(V))
    return (out_emb, out_bias)
